# Initial kernel scaffold; baseline (speedup 1.0000x reference)
import jax
import jax.numpy as jnp
from jax.experimental import pallas as pl


def kernel(x, edge_index, action, W1, b1, W2, b2, Wl1, bl1, Wl2, bl2):
    raise NotImplementedError("write your pallas kernel here")


# speedup vs baseline: 5.6963x; 5.6963x over previous
import functools

import jax
import jax.numpy as jnp
from jax import lax
from jax.experimental import pallas as pl
from jax.experimental.pallas import tpu as pltpu
from jax.experimental.pallas import tpu_sc as plsc

NN = 10000
NP = 10240
EE = 160000
EP = 163840
DIN, DH, DA = 256, 512, 16
NCB = 4
CB = 128
NTILE = 16
RPT = NP // NTILE
ECH = 80

@functools.cache
def _mesh():
    return plsc.VectorSubcoreMesh(core_axis_name="c", subcore_axis_name="s",
                                  num_cores=2, num_subcores=NTILE)


def _sc_deg_body(dst32, zeros1, deg_out, idx_v, ones_v, acc, _sem):
    c = lax.axis_index("c")
    s = lax.axis_index("s")
    wid = c * NTILE + s
    pltpu.sync_copy(dst32.at[wid], idx_v)
    for k in range(8):
        ones_v[pl.ds(k * 16, 16)] = jnp.ones((16,), jnp.float32)
    pltpu.sync_copy(zeros1, acc.at[pl.ds(s * RPT, RPT)])
    plsc.subcore_barrier()

    def body(j, carry):
        pltpu.sync_copy(ones_v, acc.at[idx_v.at[j]], add=True)
        return carry

    lax.fori_loop(0, 40, body, 0)
    plsc.subcore_barrier()
    pltpu.sync_copy(acc.at[pl.ds(s * RPT, RPT)],
                    deg_out.at[c, pl.ds(s * RPT, RPT)])


@functools.cache
def _sc_deg():
    return pl.kernel(
        _sc_deg_body,
        out_type=jax.ShapeDtypeStruct((2, NP), jnp.float32),
        mesh=_mesh(),
        scratch_types=[
            pltpu.VMEM((40, 128), jnp.int32),
            pltpu.VMEM((128,), jnp.float32),
            pltpu.VMEM_SHARED((NP,), jnp.float32),
            pltpu.SemaphoreType.DMA,
        ],
    )


def _sc_prop_body(src4, dst16, g_hbm, zeros2, t_out,
                  src_t, dst_t, gbuf, acc, _sem):
    c = lax.axis_index("c")
    s = lax.axis_index("s")
    pltpu.sync_copy(dst16.at[s], dst_t)
    for cb in range(2):
        cbg = 2 * c + cb
        pltpu.sync_copy(src4.at[cbg * NTILE + s], src_t)
        pltpu.sync_copy(zeros2, acc.at[pl.ds(s * RPT, RPT)])
        plsc.subcore_barrier()

        def body(j, carry):
            pltpu.async_copy(g_hbm.at[src_t.at[j]], gbuf, _sem).wait()
            pltpu.sync_copy(gbuf, acc.at[dst_t.at[j]], add=True)
            return carry

        lax.fori_loop(0, ECH, body, 0)
        plsc.subcore_barrier()
        pltpu.sync_copy(acc.at[pl.ds(s * RPT, RPT)],
                        t_out.at[pl.ds(cbg * NP + s * RPT, RPT)])


@functools.cache
def _sc_prop():
    return pl.kernel(
        _sc_prop_body,
        out_type=jax.ShapeDtypeStruct((NCB * NP, CB), jnp.float32),
        mesh=_mesh(),
        scratch_types=[
            pltpu.VMEM((ECH, 128), jnp.int32),
            pltpu.VMEM((ECH, 128), jnp.int32),
            pltpu.VMEM((128, CB), jnp.float32),
            pltpu.VMEM_SHARED((NP, CB), jnp.float32),
            pltpu.SemaphoreType.DMA,
        ],
    )


_RB = 1024
_GRID = NP // _RB


def _dis_of(deg2_blk):
    return lax.rsqrt(deg2_blk[0, :] + deg2_blk[1, :] + 1.0)


def _tc_a_body(x_ref, w1_ref, deg_ref, out_ref):
    dis = _dis_of(deg_ref[...])
    g = jnp.dot(x_ref[...], w1_ref[...], preferred_element_type=jnp.float32)
    g = g * dis[:, None]
    for cb in range(NCB):
        out_ref[cb] = g[:, cb * CB:(cb + 1) * CB]


def _tc_a(x_p, W1, deg2):
    return pl.pallas_call(
        _tc_a_body,
        grid=(_GRID,),
        in_specs=[
            pl.BlockSpec((_RB, DIN), lambda i: (i, 0)),
            pl.BlockSpec((DIN, DH), lambda i: (0, 0)),
            pl.BlockSpec((2, _RB), lambda i: (0, i)),
        ],
        out_specs=pl.BlockSpec((NCB, _RB, CB), lambda i: (0, i, 0)),
        out_shape=jax.ShapeDtypeStruct((NCB, NP, CB), jnp.float32),
    )(x_p, W1, deg2)


def _tc_c_body(t1_ref, g1_ref, deg_ref, w2_ref, b1_ref, out_ref):
    dis = _dis_of(deg_ref[...])
    cols = [dis[:, None] * (t1_ref[cb] + g1_ref[cb]) for cb in range(NCB)]
    h1 = jax.nn.relu(jnp.concatenate(cols, axis=1) + b1_ref[...])
    g2 = jnp.dot(h1, w2_ref[...], preferred_element_type=jnp.float32)
    g2 = g2 * dis[:, None]
    for cb in range(NCB):
        out_ref[cb] = g2[:, cb * CB:(cb + 1) * CB]


def _tc_c(T1, G1, deg2, W2, b1):
    return pl.pallas_call(
        _tc_c_body,
        grid=(_GRID,),
        in_specs=[
            pl.BlockSpec((NCB, _RB, CB), lambda i: (0, i, 0)),
            pl.BlockSpec((NCB, _RB, CB), lambda i: (0, i, 0)),
            pl.BlockSpec((2, _RB), lambda i: (0, i)),
            pl.BlockSpec((DH, DH), lambda i: (0, 0)),
            pl.BlockSpec((1, DH), lambda i: (0, 0)),
        ],
        out_specs=pl.BlockSpec((NCB, _RB, CB), lambda i: (0, i, 0)),
        out_shape=jax.ShapeDtypeStruct((NCB, NP, CB), jnp.float32),
    )(T1, G1, deg2, W2, b1)


def _tc_d_body(t2_ref, g2_ref, deg_ref, act_ref, wl1a_ref, wl1b_ref,
               bl1_ref, b2_ref, wl2_ref, bl2_ref, out_ref, acc_ref):
    i = pl.program_id(0)
    dis = _dis_of(deg_ref[...])
    cols = [dis[:, None] * (t2_ref[cb] + g2_ref[cb]) for cb in range(NCB)]
    h2 = jax.nn.relu(jnp.concatenate(cols, axis=1) + b2_ref[...])
    t = jnp.dot(h2, wl1a_ref[...], preferred_element_type=jnp.float32)
    t = t + jnp.dot(act_ref[...], wl1b_ref[...],
                    preferred_element_type=jnp.float32)
    t = jax.nn.relu(t + bl1_ref[...])
    q = jnp.sum(t * wl2_ref[...], axis=1)
    row = i * _RB + lax.broadcasted_iota(jnp.int32, (_RB,), 0)
    q = jnp.where(row < NN, q, 0.0)
    part = jnp.sum(q)

    @pl.when(i == 0)
    def _():
        acc_ref[0] = part

    @pl.when(i > 0)
    def _():
        acc_ref[0] = acc_ref[0] + part

    @pl.when(i == _GRID - 1)
    def _():
        out_ref[...] = jnp.full((1, 1), acc_ref[0] / NN + bl2_ref[0])


def _tc_d(T2, G2, deg2, act_p, Wl1a, Wl1b, bl1, b2, wl2row, bl2):
    return pl.pallas_call(
        _tc_d_body,
        grid=(_GRID,),
        in_specs=[
            pl.BlockSpec((NCB, _RB, CB), lambda i: (0, i, 0)),
            pl.BlockSpec((NCB, _RB, CB), lambda i: (0, i, 0)),
            pl.BlockSpec((2, _RB), lambda i: (0, i)),
            pl.BlockSpec((_RB, DA), lambda i: (i, 0)),
            pl.BlockSpec((DH, DH), lambda i: (0, 0)),
            pl.BlockSpec((DA, DH), lambda i: (0, 0)),
            pl.BlockSpec((1, DH), lambda i: (0, 0)),
            pl.BlockSpec((1, DH), lambda i: (0, 0)),
            pl.BlockSpec((1, DH), lambda i: (0, 0)),
            pl.BlockSpec(memory_space=pltpu.SMEM),
        ],
        out_specs=pl.BlockSpec((1, 1), lambda i: (0, 0)),
        out_shape=jax.ShapeDtypeStruct((1, 1), jnp.float32),
        scratch_shapes=[pltpu.SMEM((1,), jnp.float32)],
    )(T2, G2, deg2, act_p, Wl1a, Wl1b, bl1, b2, wl2row, bl2)


@jax.jit
def kernel(x, edge_index, action, W1, b1, W2, b2, Wl1, bl1, Wl2, bl2):
    f32 = jnp.float32
    src = edge_index[0]
    dst = edge_index[1]
    pad = jnp.full((EP - EE,), NN, jnp.int32)
    srcp = jnp.concatenate([src, pad])
    dstp = jnp.concatenate([dst, pad])
    src4 = (srcp.reshape(1, NTILE, ECH, 128)
            + (jnp.arange(NCB, dtype=jnp.int32) * NP).reshape(NCB, 1, 1, 1)
            ).reshape(NCB * NTILE, ECH, 128)
    dst16 = dstp.reshape(NTILE, ECH, 128)
    dst32 = dstp.reshape(32, 40, 128)

    x_p = jnp.concatenate([x, jnp.zeros((NP - NN, DIN), f32)])
    act_p = jnp.concatenate([action, jnp.zeros((NP - NN, DA), f32)])
    zeros1 = jnp.zeros((RPT,), f32)
    zeros2 = jnp.zeros((RPT, CB), f32)

    deg2 = _sc_deg()(dst32, zeros1)
    G1 = _tc_a(x_p, W1, deg2)
    T1 = _sc_prop()(src4, dst16, G1.reshape(NCB * NP, CB), zeros2)
    G2 = _tc_c(T1.reshape(NCB, NP, CB), G1, deg2, W2, b1.reshape(1, DH))
    T2 = _sc_prop()(src4, dst16, G2.reshape(NCB * NP, CB), zeros2)
    q = _tc_d(T2.reshape(NCB, NP, CB), G2, deg2, act_p,
              Wl1[:DH, :], Wl1[DH:, :], bl1.reshape(1, DH),
              b2.reshape(1, DH), Wl2.reshape(1, DH), bl2)
    return q


# speedup vs baseline: 7.0145x; 1.2314x over previous
import functools

import jax
import jax.numpy as jnp
from jax import lax
from jax.experimental import pallas as pl
from jax.experimental.pallas import tpu as pltpu
from jax.experimental.pallas import tpu_sc as plsc

NN = 10000
NP = 10240
EE = 160000
EP = 163840
DIN, DH, DA = 256, 512, 16
NCB = 4
CB = 128
NTILE = 16
RPT = NP // NTILE
ECH = 80
NCHK = EP // NTILE // 128

@functools.cache
def _mesh():
    return plsc.VectorSubcoreMesh(core_axis_name="c", subcore_axis_name="s",
                                  num_cores=2, num_subcores=NTILE)


def _sc_deg_body(dst32, zeros1, deg_out, idx_v, ones_v, acc, _sem):
    c = lax.axis_index("c")
    s = lax.axis_index("s")
    wid = c * NTILE + s
    pltpu.sync_copy(dst32.at[wid], idx_v)
    for k in range(8):
        ones_v[pl.ds(k * 16, 16)] = jnp.ones((16,), jnp.float32)
    pltpu.sync_copy(zeros1, acc.at[pl.ds(s * RPT, RPT)])
    plsc.subcore_barrier()

    def body(j, carry):
        pltpu.sync_copy(ones_v, acc.at[idx_v.at[j]], add=True)
        return carry

    lax.fori_loop(0, 40, body, 0)
    plsc.subcore_barrier()
    pltpu.sync_copy(acc.at[pl.ds(s * RPT, RPT)],
                    deg_out.at[c, pl.ds(s * RPT, RPT)])


@functools.cache
def _sc_deg():
    return pl.kernel(
        _sc_deg_body,
        out_type=jax.ShapeDtypeStruct((2, NP), jnp.float32),
        mesh=_mesh(),
        scratch_types=[
            pltpu.VMEM((40, 128), jnp.int32),
            pltpu.VMEM((128,), jnp.float32),
            pltpu.VMEM_SHARED((NP,), jnp.float32),
            pltpu.SemaphoreType.DMA,
        ],
    )


def _sc_prop_body(src16, dst16, g_hbm, t_out,
                  dst_t, sbuf, buf_a, buf_b, acc,
                  se0, se1, se2, se3, sga, sgb, ssa, ssb):
    c = lax.axis_index("c")
    s = lax.axis_index("s")
    src_me = src16.at[s]
    se = [se0, se1, se2, se3]
    dbuf = [buf_a, buf_b]
    sg = [sga, sgb]
    ss = [ssa, ssb]
    pltpu.sync_copy(dst16.at[s], dst_t)

    def fetch_idx(j, k):
        pltpu.async_copy(src_me.at[j], sbuf.at[k], se[k])

    def wait_idx(k):
        pltpu.make_async_copy(src_me.at[0], sbuf.at[k], se[k]).wait()

    for cb in range(2):
        cbg = 2 * c + cb
        off = cbg * NP

        def zrow(i, carry):
            for k in range(CB // 16):
                buf_a[i, pl.ds(k * 16, 16)] = jnp.zeros((16,), jnp.float32)
            return carry

        lax.fori_loop(0, 128, zrow, 0)

        def zcp(r, carry):
            pltpu.sync_copy(buf_a, acc.at[pl.ds(s * RPT + r * 128, 128)])
            return carry

        lax.fori_loop(0, RPT // 128, zcp, 0)
        for k in range(3):
            fetch_idx(k, k)
        plsc.subcore_barrier()

        def chunk(j, k, d):
            wait_idx(k)
            for t in range(128 // 16):
                sl = pl.ds(t * 16, 16)
                sbuf[k, sl] = sbuf[k, sl] + off

            @pl.when(j >= 2)
            def _():
                pltpu.make_async_copy(dbuf[d], acc.at[dst_t.at[0]],
                                      ss[d]).wait()

            pltpu.async_copy(g_hbm.at[sbuf.at[k]], dbuf[d], sg[d])

            @pl.when(j >= 1)
            def _():
                pltpu.make_async_copy(g_hbm.at[sbuf.at[k]], dbuf[1 - d],
                                      sg[1 - d]).wait()
                pltpu.async_copy(dbuf[1 - d], acc.at[dst_t.at[j - 1]],
                                 ss[1 - d], add=True)

            @pl.when(j + 3 < NCHK)
            def _():
                fetch_idx(j + 3, (k + 3) % 4)

        def body(m, carry):
            j0 = 4 * m
            for t in range(4):
                chunk(j0 + t, t, t % 2)
            return carry

        lax.fori_loop(0, NCHK // 4, body, 0)
        pltpu.make_async_copy(g_hbm.at[sbuf.at[3]], dbuf[1], sg[1]).wait()
        pltpu.async_copy(dbuf[1], acc.at[dst_t.at[NCHK - 1]], ss[1],
                         add=True)
        pltpu.make_async_copy(dbuf[0], acc.at[dst_t.at[0]], ss[0]).wait()
        pltpu.make_async_copy(dbuf[1], acc.at[dst_t.at[0]], ss[1]).wait()
        plsc.subcore_barrier()
        pltpu.sync_copy(acc.at[pl.ds(s * RPT, RPT)],
                        t_out.at[pl.ds(cbg * NP + s * RPT, RPT)])


@functools.cache
def _sc_prop():
    return pl.kernel(
        _sc_prop_body,
        out_type=jax.ShapeDtypeStruct((NCB * NP, CB), jnp.float32),
        mesh=_mesh(),
        scratch_types=[
            pltpu.VMEM((NCHK, 128), jnp.int32),
            pltpu.VMEM((4, 128), jnp.int32),
            pltpu.VMEM((128, CB), jnp.float32),
            pltpu.VMEM((128, CB), jnp.float32),
            pltpu.VMEM_SHARED((NP, CB), jnp.float32),
        ] + [pltpu.SemaphoreType.DMA] * 8,
    )


_RB = 1024
_GRID = NP // _RB


def _dis_of(deg2_blk):
    return lax.rsqrt(deg2_blk[0, :] + deg2_blk[1, :] + 1.0)


def _tc_a_body(x_ref, w1_ref, deg_ref, out_ref):
    dis = _dis_of(deg_ref[...])
    g = jnp.dot(x_ref[...], w1_ref[...], preferred_element_type=jnp.float32)
    g = g * dis[:, None]
    for cb in range(NCB):
        out_ref[cb] = g[:, cb * CB:(cb + 1) * CB]


def _tc_a(x_p, W1, deg2):
    return pl.pallas_call(
        _tc_a_body,
        grid=(_GRID,),
        in_specs=[
            pl.BlockSpec((_RB, DIN), lambda i: (i, 0)),
            pl.BlockSpec((DIN, DH), lambda i: (0, 0)),
            pl.BlockSpec((2, _RB), lambda i: (0, i)),
        ],
        out_specs=pl.BlockSpec((NCB, _RB, CB), lambda i: (0, i, 0)),
        out_shape=jax.ShapeDtypeStruct((NCB, NP, CB), jnp.float32),
    )(x_p, W1, deg2)


def _tc_c_body(t1_ref, g1_ref, deg_ref, w2_ref, b1_ref, out_ref):
    dis = _dis_of(deg_ref[...])
    cols = [dis[:, None] * (t1_ref[cb] + g1_ref[cb]) for cb in range(NCB)]
    h1 = jax.nn.relu(jnp.concatenate(cols, axis=1) + b1_ref[...])
    g2 = jnp.dot(h1, w2_ref[...], preferred_element_type=jnp.float32)
    g2 = g2 * dis[:, None]
    for cb in range(NCB):
        out_ref[cb] = g2[:, cb * CB:(cb + 1) * CB]


def _tc_c(T1, G1, deg2, W2, b1):
    return pl.pallas_call(
        _tc_c_body,
        grid=(_GRID,),
        in_specs=[
            pl.BlockSpec((NCB, _RB, CB), lambda i: (0, i, 0)),
            pl.BlockSpec((NCB, _RB, CB), lambda i: (0, i, 0)),
            pl.BlockSpec((2, _RB), lambda i: (0, i)),
            pl.BlockSpec((DH, DH), lambda i: (0, 0)),
            pl.BlockSpec((1, DH), lambda i: (0, 0)),
        ],
        out_specs=pl.BlockSpec((NCB, _RB, CB), lambda i: (0, i, 0)),
        out_shape=jax.ShapeDtypeStruct((NCB, NP, CB), jnp.float32),
    )(T1, G1, deg2, W2, b1)


def _tc_d_body(t2_ref, g2_ref, deg_ref, act_ref, wl1a_ref, wl1b_ref,
               bl1_ref, b2_ref, wl2_ref, bl2_ref, out_ref, acc_ref):
    i = pl.program_id(0)
    dis = _dis_of(deg_ref[...])
    cols = [dis[:, None] * (t2_ref[cb] + g2_ref[cb]) for cb in range(NCB)]
    h2 = jax.nn.relu(jnp.concatenate(cols, axis=1) + b2_ref[...])
    t = jnp.dot(h2, wl1a_ref[...], preferred_element_type=jnp.float32)
    t = t + jnp.dot(act_ref[...], wl1b_ref[...],
                    preferred_element_type=jnp.float32)
    t = jax.nn.relu(t + bl1_ref[...])
    q = jnp.sum(t * wl2_ref[...], axis=1)
    row = i * _RB + lax.broadcasted_iota(jnp.int32, (_RB,), 0)
    q = jnp.where(row < NN, q, 0.0)
    part = jnp.sum(q)

    @pl.when(i == 0)
    def _():
        acc_ref[0] = part

    @pl.when(i > 0)
    def _():
        acc_ref[0] = acc_ref[0] + part

    @pl.when(i == _GRID - 1)
    def _():
        out_ref[...] = jnp.full((1, 1), acc_ref[0] / NN + bl2_ref[0])


def _tc_d(T2, G2, deg2, act_p, Wl1a, Wl1b, bl1, b2, wl2row, bl2):
    return pl.pallas_call(
        _tc_d_body,
        grid=(_GRID,),
        in_specs=[
            pl.BlockSpec((NCB, _RB, CB), lambda i: (0, i, 0)),
            pl.BlockSpec((NCB, _RB, CB), lambda i: (0, i, 0)),
            pl.BlockSpec((2, _RB), lambda i: (0, i)),
            pl.BlockSpec((_RB, DA), lambda i: (i, 0)),
            pl.BlockSpec((DH, DH), lambda i: (0, 0)),
            pl.BlockSpec((DA, DH), lambda i: (0, 0)),
            pl.BlockSpec((1, DH), lambda i: (0, 0)),
            pl.BlockSpec((1, DH), lambda i: (0, 0)),
            pl.BlockSpec((1, DH), lambda i: (0, 0)),
            pl.BlockSpec(memory_space=pltpu.SMEM),
        ],
        out_specs=pl.BlockSpec((1, 1), lambda i: (0, 0)),
        out_shape=jax.ShapeDtypeStruct((1, 1), jnp.float32),
        scratch_shapes=[pltpu.SMEM((1,), jnp.float32)],
    )(T2, G2, deg2, act_p, Wl1a, Wl1b, bl1, b2, wl2row, bl2)


@jax.jit
def kernel(x, edge_index, action, W1, b1, W2, b2, Wl1, bl1, Wl2, bl2):
    f32 = jnp.float32
    src = edge_index[0]
    dst = edge_index[1]
    pad = jnp.full((EP - EE,), NN, jnp.int32)
    srcp = jnp.concatenate([src, pad])
    dstp = jnp.concatenate([dst, pad])
    src16 = srcp.reshape(NTILE, NCHK, 128)
    dst16 = dstp.reshape(NTILE, NCHK, 128)
    dst32 = dstp.reshape(32, 40, 128)

    x_p = jnp.concatenate([x, jnp.zeros((NP - NN, DIN), f32)])
    act_p = jnp.concatenate([action, jnp.zeros((NP - NN, DA), f32)])
    zeros1 = jnp.zeros((RPT,), f32)

    deg2 = _sc_deg()(dst32, zeros1)
    G1 = _tc_a(x_p, W1, deg2)
    T1 = _sc_prop()(src16, dst16, G1.reshape(NCB * NP, CB))
    G2 = _tc_c(T1.reshape(NCB, NP, CB), G1, deg2, W2, b1.reshape(1, DH))
    T2 = _sc_prop()(src16, dst16, G2.reshape(NCB * NP, CB))
    q = _tc_d(T2.reshape(NCB, NP, CB), G2, deg2, act_p,
              Wl1[:DH, :], Wl1[DH:, :], bl1.reshape(1, DH),
              b2.reshape(1, DH), Wl2.reshape(1, DH), bl2)
    return q


# speedup vs baseline: 8.6604x; 1.2347x over previous
import functools

import jax
import jax.numpy as jnp
from jax import lax
from jax.experimental import pallas as pl
from jax.experimental.pallas import tpu as pltpu
from jax.experimental.pallas import tpu_sc as plsc

NN = 10000
NP = 10240
EE = 160000
EP = 163840
DIN, DH, DA = 256, 512, 16
NCB = 4
NCBX = 2
CB = 128
NTILE = 16
RPT = NP // NTILE
NCHK = EP // NTILE // 128

@functools.cache
def _mesh():
    return plsc.VectorSubcoreMesh(core_axis_name="c", subcore_axis_name="s",
                                  num_cores=2, num_subcores=NTILE)


def _sc_deg_body(dst32, zeros1, deg_out, idx_v, ones_v, acc, _sem):
    c = lax.axis_index("c")
    s = lax.axis_index("s")
    wid = c * NTILE + s
    pltpu.sync_copy(dst32.at[wid], idx_v)
    for k in range(8):
        ones_v[pl.ds(k * 16, 16)] = jnp.ones((16,), jnp.float32)
    pltpu.sync_copy(zeros1, acc.at[pl.ds(s * RPT, RPT)])
    plsc.subcore_barrier()

    def body(j, carry):
        pltpu.sync_copy(ones_v, acc.at[idx_v.at[j]], add=True)
        return carry

    lax.fori_loop(0, 40, body, 0)
    plsc.subcore_barrier()
    pltpu.sync_copy(acc.at[pl.ds(s * RPT, RPT)],
                    deg_out.at[c, pl.ds(s * RPT, RPT)])


@functools.cache
def _sc_deg():
    return pl.kernel(
        _sc_deg_body,
        out_type=jax.ShapeDtypeStruct((2, NP), jnp.float32),
        mesh=_mesh(),
        scratch_types=[
            pltpu.VMEM((40, 128), jnp.int32),
            pltpu.VMEM((128,), jnp.float32),
            pltpu.VMEM_SHARED((NP,), jnp.float32),
            pltpu.SemaphoreType.DMA,
        ],
    )


def _sc_prop_body(nblk, src16, dst16, g_hbm, t_out,
                  dst_t, sbuf, buf_a, buf_b, acc,
                  se0, se1, se2, se3, sga, sgb, ssa, ssb):
    c = lax.axis_index("c")
    s = lax.axis_index("s")
    src_me = src16.at[s]
    se = [se0, se1, se2, se3]
    dbuf = [buf_a, buf_b]
    sg = [sga, sgb]
    ss = [ssa, ssb]
    pltpu.sync_copy(dst16.at[s], dst_t)

    def fetch_idx(j, k):
        pltpu.async_copy(src_me.at[j], sbuf.at[k], se[k])

    def wait_idx(k):
        pltpu.make_async_copy(src_me.at[0], sbuf.at[k], se[k]).wait()

    for cb in range(nblk):
        cbg = nblk * c + cb
        off = cbg * NP

        def zrow(i, carry):
            for k in range(CB // 16):
                buf_a[i, pl.ds(k * 16, 16)] = jnp.zeros((16,), jnp.float32)
            return carry

        lax.fori_loop(0, 128, zrow, 0)

        def zcp(r, carry):
            pltpu.sync_copy(buf_a, acc.at[pl.ds(s * RPT + r * 128, 128)])
            return carry

        lax.fori_loop(0, RPT // 128, zcp, 0)
        for k in range(3):
            fetch_idx(k, k)
        plsc.subcore_barrier()

        def chunk(j, k, d):
            wait_idx(k)
            for t in range(128 // 16):
                sl = pl.ds(t * 16, 16)
                sbuf[k, sl] = sbuf[k, sl] + off

            @pl.when(j >= 2)
            def _():
                pltpu.make_async_copy(dbuf[d], acc.at[dst_t.at[0]],
                                      ss[d]).wait()

            pltpu.async_copy(g_hbm.at[sbuf.at[k]], dbuf[d], sg[d])

            @pl.when(j >= 1)
            def _():
                pltpu.make_async_copy(g_hbm.at[sbuf.at[k]], dbuf[1 - d],
                                      sg[1 - d]).wait()
                pltpu.async_copy(dbuf[1 - d], acc.at[dst_t.at[j - 1]],
                                 ss[1 - d], add=True)

            @pl.when(j + 3 < NCHK)
            def _():
                fetch_idx(j + 3, (k + 3) % 4)

        def body(m, carry):
            j0 = 4 * m
            for t in range(4):
                chunk(j0 + t, t, t % 2)
            return carry

        lax.fori_loop(0, NCHK // 4, body, 0)
        pltpu.make_async_copy(g_hbm.at[sbuf.at[3]], dbuf[1], sg[1]).wait()
        pltpu.async_copy(dbuf[1], acc.at[dst_t.at[NCHK - 1]], ss[1],
                         add=True)
        pltpu.make_async_copy(dbuf[0], acc.at[dst_t.at[0]], ss[0]).wait()
        pltpu.make_async_copy(dbuf[1], acc.at[dst_t.at[0]], ss[1]).wait()
        plsc.subcore_barrier()
        pltpu.sync_copy(acc.at[pl.ds(s * RPT, RPT)],
                        t_out.at[pl.ds(cbg * NP + s * RPT, RPT)])


@functools.cache
def _sc_prop(nblk):
    return pl.kernel(
        functools.partial(_sc_prop_body, nblk),
        out_type=jax.ShapeDtypeStruct((2 * nblk * NP, CB), jnp.float32),
        mesh=_mesh(),
        scratch_types=[
            pltpu.VMEM((NCHK, 128), jnp.int32),
            pltpu.VMEM((4, 128), jnp.int32),
            pltpu.VMEM((128, CB), jnp.float32),
            pltpu.VMEM((128, CB), jnp.float32),
            pltpu.VMEM_SHARED((NP, CB), jnp.float32),
        ] + [pltpu.SemaphoreType.DMA] * 8,
    )


_RB = 1024
_GRID = NP // _RB


def _dis_of(deg2_blk):
    return lax.rsqrt(deg2_blk[0, :] + deg2_blk[1, :] + 1.0)


def _tc_a_body(x_ref, deg_ref, out_ref):
    dis = _dis_of(deg_ref[...])
    g = x_ref[...] * dis[:, None]
    for cb in range(NCBX):
        out_ref[cb] = g[:, cb * CB:(cb + 1) * CB]


def _tc_a(x_p, deg2):
    return pl.pallas_call(
        _tc_a_body,
        grid=(_GRID,),
        in_specs=[
            pl.BlockSpec((_RB, DIN), lambda i: (i, 0)),
            pl.BlockSpec((2, _RB), lambda i: (0, i)),
        ],
        out_specs=pl.BlockSpec((NCBX, _RB, CB), lambda i: (0, i, 0)),
        out_shape=jax.ShapeDtypeStruct((NCBX, NP, CB), jnp.float32),
    )(x_p, deg2)


def _tc_c_body(t1_ref, g1_ref, deg_ref, w1_ref, w2_ref, b1_ref, out_ref):
    dis = _dis_of(deg_ref[...])
    cols = [dis[:, None] * (t1_ref[cb] + g1_ref[cb]) for cb in range(NCBX)]
    z = jnp.concatenate(cols, axis=1)
    h1 = jnp.dot(z, w1_ref[...], preferred_element_type=jnp.float32,
                 precision=lax.Precision.HIGHEST)
    h1 = jax.nn.relu(h1 + b1_ref[...])
    g2 = jnp.dot(h1, w2_ref[...], preferred_element_type=jnp.float32,
                 precision=lax.Precision.HIGHEST)
    g2 = g2 * dis[:, None]
    for cb in range(NCB):
        out_ref[cb] = g2[:, cb * CB:(cb + 1) * CB]


def _tc_c(T1, G1, deg2, W1, W2, b1):
    return pl.pallas_call(
        _tc_c_body,
        grid=(_GRID,),
        in_specs=[
            pl.BlockSpec((NCBX, _RB, CB), lambda i: (0, i, 0)),
            pl.BlockSpec((NCBX, _RB, CB), lambda i: (0, i, 0)),
            pl.BlockSpec((2, _RB), lambda i: (0, i)),
            pl.BlockSpec((DIN, DH), lambda i: (0, 0)),
            pl.BlockSpec((DH, DH), lambda i: (0, 0)),
            pl.BlockSpec((1, DH), lambda i: (0, 0)),
        ],
        out_specs=pl.BlockSpec((NCB, _RB, CB), lambda i: (0, i, 0)),
        out_shape=jax.ShapeDtypeStruct((NCB, NP, CB), jnp.float32),
    )(T1, G1, deg2, W1, W2, b1)


def _tc_d_body(t2_ref, g2_ref, deg_ref, act_ref, wl1a_ref, wl1b_ref,
               bl1_ref, b2_ref, wl2_ref, bl2_ref, out_ref, acc_ref):
    i = pl.program_id(0)
    dis = _dis_of(deg_ref[...])
    cols = [dis[:, None] * (t2_ref[cb] + g2_ref[cb]) for cb in range(NCB)]
    h2 = jax.nn.relu(jnp.concatenate(cols, axis=1) + b2_ref[...])
    t = jnp.dot(h2, wl1a_ref[...], preferred_element_type=jnp.float32)
    t = t + jnp.dot(act_ref[...], wl1b_ref[...],
                    preferred_element_type=jnp.float32)
    t = jax.nn.relu(t + bl1_ref[...])
    q = jnp.sum(t * wl2_ref[...], axis=1)
    row = i * _RB + lax.broadcasted_iota(jnp.int32, (_RB,), 0)
    q = jnp.where(row < NN, q, 0.0)
    part = jnp.sum(q)

    @pl.when(i == 0)
    def _():
        acc_ref[0] = part

    @pl.when(i > 0)
    def _():
        acc_ref[0] = acc_ref[0] + part

    @pl.when(i == _GRID - 1)
    def _():
        out_ref[...] = jnp.full((1, 1), acc_ref[0] / NN + bl2_ref[0])


def _tc_d(T2, G2, deg2, act_p, Wl1a, Wl1b, bl1, b2, wl2row, bl2):
    return pl.pallas_call(
        _tc_d_body,
        grid=(_GRID,),
        in_specs=[
            pl.BlockSpec((NCB, _RB, CB), lambda i: (0, i, 0)),
            pl.BlockSpec((NCB, _RB, CB), lambda i: (0, i, 0)),
            pl.BlockSpec((2, _RB), lambda i: (0, i)),
            pl.BlockSpec((_RB, DA), lambda i: (i, 0)),
            pl.BlockSpec((DH, DH), lambda i: (0, 0)),
            pl.BlockSpec((DA, DH), lambda i: (0, 0)),
            pl.BlockSpec((1, DH), lambda i: (0, 0)),
            pl.BlockSpec((1, DH), lambda i: (0, 0)),
            pl.BlockSpec((1, DH), lambda i: (0, 0)),
            pl.BlockSpec(memory_space=pltpu.SMEM),
        ],
        out_specs=pl.BlockSpec((1, 1), lambda i: (0, 0)),
        out_shape=jax.ShapeDtypeStruct((1, 1), jnp.float32),
        scratch_shapes=[pltpu.SMEM((1,), jnp.float32)],
    )(T2, G2, deg2, act_p, Wl1a, Wl1b, bl1, b2, wl2row, bl2)


@jax.jit
def kernel(x, edge_index, action, W1, b1, W2, b2, Wl1, bl1, Wl2, bl2):
    f32 = jnp.float32
    src = edge_index[0]
    dst = edge_index[1]
    pad = jnp.full((EP - EE,), NN, jnp.int32)
    srcp = jnp.concatenate([src, pad])
    dstp = jnp.concatenate([dst, pad])
    src16 = srcp.reshape(NTILE, NCHK, 128)
    dst16 = dstp.reshape(NTILE, NCHK, 128)
    dst32 = dstp.reshape(32, 40, 128)

    x_p = jnp.concatenate([x, jnp.zeros((NP - NN, DIN), f32)])
    act_p = jnp.concatenate([action, jnp.zeros((NP - NN, DA), f32)])
    zeros1 = jnp.zeros((RPT,), f32)

    deg2 = _sc_deg()(dst32, zeros1)
    G1 = _tc_a(x_p, deg2)
    T1 = _sc_prop(1)(src16, dst16, G1.reshape(NCBX * NP, CB))
    G2 = _tc_c(T1.reshape(NCBX, NP, CB), G1, deg2, W1, W2,
               b1.reshape(1, DH))
    T2 = _sc_prop(2)(src16, dst16, G2.reshape(NCB * NP, CB))
    q = _tc_d(T2.reshape(NCB, NP, CB), G2, deg2, act_p,
              Wl1[:DH, :], Wl1[DH:, :], bl1.reshape(1, DH),
              b2.reshape(1, DH), Wl2.reshape(1, DH), bl2)
    return q
